# baseline (speedup 1.0000x reference)
import functools

import jax
import jax.numpy as jnp
from jax.experimental import pallas as pl
from jax.experimental.pallas import tpu as pltpu

CH = 400
NBUF = 3


def _body(x_ref, wt_ref, b_ref, adj_hbm, out_ref, buf, sem, *, n, d_out):
    nchunks = n // CH
    wt = wt_ref[...]
    bias = b_ref[...]

    def copy(c, slot):
        return pltpu.make_async_copy(
            adj_hbm.at[pl.ds(c * CH, CH), :],
            buf.at[slot],
            sem.at[slot],
        )

    for c in range(NBUF - 1):
        copy(c, c).start()

    def loop(c, carry):
        slot = jax.lax.rem(c, NBUF)
        copy(c, slot).wait()
        nxt = c + NBUF - 1

        @pl.when(nxt < nchunks)
        def _next():
            copy(nxt, jax.lax.rem(nxt, NBUF)).start()

        a_bf = buf[slot].astype(jnp.bfloat16)
        h = jnp.dot(a_bf, x_ref[...], preferred_element_type=jnp.float32)
        out_ref[pl.ds(c * CH, CH), :] = (
            jnp.dot(h, wt, preferred_element_type=jnp.float32) + bias
        )
        return carry

    jax.lax.fori_loop(0, nchunks, loop, 0)


def kernel(x, adj, W, b):
    n, d_in = x.shape
    d_out = W.shape[0]
    x_bf = x.astype(jnp.bfloat16)
    wt = W.T
    b2 = b.reshape(1, d_out)
    return pl.pallas_call(
        functools.partial(_body, n=n, d_out=d_out),
        in_specs=[
            pl.BlockSpec(memory_space=pltpu.MemorySpace.VMEM),
            pl.BlockSpec(memory_space=pltpu.MemorySpace.VMEM),
            pl.BlockSpec(memory_space=pltpu.MemorySpace.VMEM),
            pl.BlockSpec(memory_space=pltpu.MemorySpace.HBM),
        ],
        out_specs=pl.BlockSpec(memory_space=pltpu.MemorySpace.VMEM),
        out_shape=jax.ShapeDtypeStruct((n, d_out), jnp.float32),
        scratch_shapes=[
            pltpu.VMEM((NBUF, CH, n), jnp.float32),
            pltpu.SemaphoreType.DMA((NBUF,)),
        ],
        compiler_params=pltpu.CompilerParams(
            vmem_limit_bytes=64 * 1024 * 1024,
        ),
    )(x_bf, wt, b2, adj)


# speedup vs baseline: 1.0633x; 1.0633x over previous
import jax
import jax.numpy as jnp
from jax.experimental import pallas as pl
from jax.experimental.pallas import tpu as pltpu


def _gcn_block(x_ref, adj_ref, w_ref, b_ref, out_ref, xw_ref):
    @pl.when(pl.program_id(0) == 0)
    def _fold_weights():
        xw_ref[...] = jax.lax.dot_general(
            x_ref[...],
            w_ref[...],
            (((1,), (1,)), ((), ())),
            preferred_element_type=jnp.float32,
        )

    h = jnp.dot(adj_ref[...], xw_ref[...], preferred_element_type=jnp.float32)
    out_ref[...] = h + b_ref[...]


def kernel(x, adj, W, b):
    n, d_in = x.shape
    d_out = W.shape[0]
    bm = 400
    b2 = b.reshape(1, d_out)
    return pl.pallas_call(
        _gcn_block,
        grid=(n // bm,),
        in_specs=[
            pl.BlockSpec((n, d_in), lambda i: (0, 0)),
            pl.BlockSpec((bm, n), lambda i: (i, 0)),
            pl.BlockSpec((d_out, d_in), lambda i: (0, 0)),
            pl.BlockSpec((1, d_out), lambda i: (0, 0)),
        ],
        out_specs=pl.BlockSpec((bm, d_out), lambda i: (i, 0)),
        out_shape=jax.ShapeDtypeStruct((n, d_out), jnp.float32),
        scratch_shapes=[pltpu.VMEM((n, d_out), jnp.float32)],
        compiler_params=pltpu.CompilerParams(
            dimension_semantics=("arbitrary",),
        ),
    )(x, adj, W, b2)
